# Initial kernel scaffold; baseline (speedup 1.0000x reference)
import jax
import jax.numpy as jnp
from jax.experimental import pallas as pl


def kernel(token_indices, embedding, Acoeff, Bbasis):
    raise NotImplementedError("write your pallas kernel here")


# speedup vs baseline: 14.4139x; 14.4139x over previous
import functools

import jax
import jax.numpy as jnp
from jax import lax
from jax.experimental import pallas as pl
from jax.experimental.pallas import tpu as pltpu
from jax.experimental.pallas import tpu_sc as plsc

N = 524288
M = 32
L = 64
NC = 2
NS = 16
NW = NC * NS
TPW = N // NW
C = 1024
NCHUNK = TPW // C
RPT = C // L
SPC = C // 128
NBUF = 3


def _sc_body(tok_hbm, emb_hbm, b2_hbm, a2_hbm, out_hbm,
             idx_v, rows_v, b2_v, a2_v, gsem, osem):
    wid = lax.axis_index("s") * NC + lax.axis_index("c")
    pltpu.sync_copy(b2_hbm, b2_v)
    pltpu.sync_copy(a2_hbm, a2_v)
    pltpu.sync_copy(
        tok_hbm.at[pl.ds(pl.multiple_of(wid * (TPW // 128), 8), TPW // 128)],
        idx_v)

    def gathers(c, b):
        for s in range(SPC):
            pltpu.async_copy(emb_hbm.at[idx_v.at[c * SPC + s]],
                             rows_v.at[b, pl.ds(s * 128, 128)], gsem.at[b])

    def wait_gathers(c, b):
        for s in range(SPC):
            pltpu.make_async_copy(emb_hbm.at[idx_v.at[c * SPC + s]],
                                  rows_v.at[b, pl.ds(s * 128, 128)],
                                  gsem.at[b]).wait()

    def out_copy(c, b):
        base = pl.multiple_of(wid * TPW + c * C, 8)
        return pltpu.make_async_copy(rows_v.at[b], out_hbm.at[pl.ds(base, C)],
                                     osem.at[b])

    def compute(b):
        def jbody(j, carry2):
            blo = b2_v[j, 0:16]
            bhi = b2_v[j, 16:32]
            alo = a2_v[j, 0:16]
            ahi = a2_v[j, 16:32]
            for r in range(RPT):
                t = j + r * L
                xlo = rows_v[b, t, 0:16]
                xhi = rows_v[b, t, 16:32]
                s = jnp.sum(blo * xlo + bhi * xhi)
                rows_v[b, t, 0:16] = alo * s
                rows_v[b, t, 16:32] = ahi * s
            return carry2

        lax.fori_loop(0, L, jbody, 0)

    gathers(0, 0)
    for c in range(NCHUNK):
        b = c % NBUF
        if c + 1 < NCHUNK:
            nb = (c + 1) % NBUF
            if c + 1 >= NBUF:
                out_copy(c + 1 - NBUF, nb).wait()
            gathers(c + 1, nb)
        wait_gathers(c, b)
        compute(b)
        out_copy(c, b).start()
    for c in range(NCHUNK - NBUF, NCHUNK):
        out_copy(c, c % NBUF).wait()


@functools.partial(jax.jit, static_argnames=())
def kernel(token_indices, embedding, Acoeff, Bbasis):
    tok = token_indices.astype(jnp.int32).reshape(N // 128, 128)
    a2 = Acoeff.T.reshape(L, M)
    mesh = plsc.VectorSubcoreMesh(core_axis_name="c", subcore_axis_name="s",
                                  num_cores=NC, num_subcores=NS)
    f = pl.kernel(
        _sc_body,
        out_type=jax.ShapeDtypeStruct((N, M), jnp.float32),
        mesh=mesh,
        compiler_params=pltpu.CompilerParams(needs_layout_passes=False,
                                             use_tc_tiling_on_sc=False),
        scratch_types=[
            pltpu.VMEM((TPW // 128, 128), jnp.int32),
            pltpu.VMEM((NBUF, C, M), jnp.float32),
            pltpu.VMEM((L, M), jnp.float32),
            pltpu.VMEM((L, M), jnp.float32),
            pltpu.SemaphoreType.DMA((NBUF,)),
            pltpu.SemaphoreType.DMA((NBUF,)),
        ],
    )
    return f(tok, embedding, Bbasis, a2)
